# Initial kernel scaffold; baseline (speedup 1.0000x reference)
import jax
import jax.numpy as jnp
from jax.experimental import pallas as pl


def kernel(adj, x, w0, b0, w1, b1, w2, b2, g0, be0, rm0, rv0, g1, be1, rm1, rv1):
    raise NotImplementedError("write your pallas kernel here")


# speedup vs baseline: 1.0961x; 1.0961x over previous
import functools

import jax
import jax.numpy as jnp
from jax import lax
from jax.experimental import pallas as pl
from jax.experimental.pallas import tpu as pltpu

BN_EPS = 1e-5
LANE = 128
NEG_INF = -1e30


def _pad_to(n, m):
    return ((n + m - 1) // m) * m


def _fused_gcn_kernel(num_k, adj_ref, x_ref, w0_ref, b0_ref, w1_ref, b1_ref,
                      w2_ref, b2_ref, out_ref, adj_bf_ref, z0_ref):
    k = pl.program_id(0)
    tk = adj_ref.shape[1]

    @pl.when(k == 0)
    def _():
        z0_ref[...] = jnp.zeros_like(z0_ref)

    a = adj_ref[...].astype(jnp.bfloat16)
    adj_bf_ref[:, pl.ds(k * tk, tk)] = a
    xk = x_ref[...].astype(jnp.bfloat16)
    z0_ref[...] += jnp.dot(a, xk, preferred_element_type=jnp.float32)

    @pl.when(k == num_k - 1)
    def _():
        adj_bf = adj_bf_ref[...]
        y0 = jnp.dot(z0_ref[...].astype(jnp.bfloat16), w0_ref[...],
                     preferred_element_type=jnp.float32) + b0_ref[...]
        h1 = jnp.maximum(y0, 0.0).astype(jnp.bfloat16)
        z1 = jnp.dot(adj_bf, h1, preferred_element_type=jnp.float32)
        y1 = jnp.dot(z1.astype(jnp.bfloat16), w1_ref[...],
                     preferred_element_type=jnp.float32) + b1_ref[...]
        h2 = jnp.maximum(y1, 0.0).astype(jnp.bfloat16)
        t2 = jnp.dot(h2, w2_ref[...],
                     preferred_element_type=jnp.float32).astype(jnp.bfloat16)
        y2 = jnp.dot(adj_bf, t2, preferred_element_type=jnp.float32) + b2_ref[...]
        m = jnp.max(y2, axis=-1, keepdims=True)
        z = y2 - m
        lse = jnp.log(jnp.sum(jnp.exp(z), axis=-1, keepdims=True))
        out_ref[...] = z - lse


def kernel(adj, x, w0, b0, w1, b1, w2, b2, g0, be0, rm0, rv0, g1, be1, rm1, rv1):
    n = x.shape[0]
    np_ = _pad_to(n, LANE)
    assert np_ == adj.shape[0], "node count must be 128-aligned for this kernel"
    c0 = x.shape[1]
    c1 = w0.shape[1]
    c2 = w1.shape[1]
    n_cls = w2.shape[1]
    c3 = _pad_to(n_cls, LANE)

    a0 = g0 * lax.rsqrt(rv0 + BN_EPS)
    w0f = (w0 * a0[None, :]).astype(jnp.bfloat16)
    b0f = (be0 + (b0 - rm0) * a0).astype(jnp.float32).reshape(1, c1)
    a1 = g1 * lax.rsqrt(rv1 + BN_EPS)
    w1f = (w1 * a1[None, :]).astype(jnp.bfloat16)
    b1f = (be1 + (b1 - rm1) * a1).astype(jnp.float32).reshape(1, c2)
    w2p = jnp.pad(w2, ((0, 0), (0, c3 - n_cls))).astype(jnp.bfloat16)
    b2p = jnp.pad(b2, (0, c3 - n_cls),
                  constant_values=NEG_INF).astype(jnp.float32).reshape(1, c3)

    tk = 256 if np_ % 256 == 0 else LANE
    num_k = np_ // tk

    out = pl.pallas_call(
        functools.partial(_fused_gcn_kernel, num_k),
        out_shape=jax.ShapeDtypeStruct((np_, c3), jnp.float32),
        grid=(num_k,),
        in_specs=[
            pl.BlockSpec((np_, tk), lambda k: (0, k)),
            pl.BlockSpec((tk, c0), lambda k: (k, 0)),
            pl.BlockSpec((c0, c1), lambda k: (0, 0)),
            pl.BlockSpec((1, c1), lambda k: (0, 0)),
            pl.BlockSpec((c1, c2), lambda k: (0, 0)),
            pl.BlockSpec((1, c2), lambda k: (0, 0)),
            pl.BlockSpec((c2, c3), lambda k: (0, 0)),
            pl.BlockSpec((1, c3), lambda k: (0, 0)),
        ],
        out_specs=pl.BlockSpec((np_, c3), lambda k: (0, 0)),
        scratch_shapes=[
            pltpu.VMEM((np_, np_), jnp.bfloat16),
            pltpu.VMEM((np_, c0), jnp.float32),
        ],
        compiler_params=pltpu.CompilerParams(
            dimension_semantics=("arbitrary",),
            vmem_limit_bytes=56 * 2 ** 20,
        ),
    )(adj, x, w0f, b0f, w1f, b1f, w2p, b2p)

    return out[:n, :n_cls]
